# baseline (speedup 1.0000x reference)
import functools

import jax
import jax.numpy as jnp
import numpy as np
from jax import lax
from jax.experimental import pallas as pl
from jax.experimental.pallas import tpu as pltpu
from jax.experimental.pallas import tpu_sc as plsc

_B = 32
_N = 8192
_D = 64
_K = 64
_TK = 512
_NCH = _N // _TK
_NC = 2


def _build_sampling_consts():
    def per_sample(k):
        k, sub = jax.random.split(k)
        first = jax.random.randint(sub, (), 0, _N)
        gs = []
        for _ in range(1, _K):
            k, sub = jax.random.split(k)
            gs.append(jax.random.gumbel(sub, (_N,), jnp.float32))
        return first.astype(jnp.int32), jnp.stack(gs)

    keys = jax.random.split(jax.random.key(42), _B)
    first, g = jax.vmap(per_sample)(keys)
    w0 = jax.nn.one_hot(first, _N, dtype=jnp.float32)
    return jnp.concatenate([w0[:, None, :], jnp.exp(2.0 * g)], axis=1)


def _chunk_copy(xt_hbm, b, ci, buf, sem):
    return pltpu.make_async_copy(
        xt_hbm.at[b, :, pl.ds(ci * _TK, _TK)], buf, sem)


def _process_chunk(buf, chunk_base, c_ref, s_ref, acc_ref):
    for db in range(4):
        cv = c_ref[pl.ds(db * 16, 16)]
        cb = [jnp.full((16,), cv[dd], jnp.float32) for dd in range(16)]

        def gbody(g, _, db=db, cb=cb):
            base = g * 16
            if db == 0:
                acc = jnp.zeros((16,), jnp.float32)
            else:
                acc = acc_ref[pl.ds(base, 16)]
            for dd in range(16):
                xv = buf[db * 16 + dd, pl.ds(base, 16)]
                d_ = xv - cb[dd]
                acc = acc + d_ * d_
            if db < 3:
                acc_ref[pl.ds(base, 16)] = acc
            else:
                so = chunk_base + base
                s_ref[pl.ds(so, 16)] = jnp.minimum(s_ref[pl.ds(so, 16)], acc)
            return 0

        lax.fori_loop(0, _TK // 16, gbody, 0)


def _body(x_hbm, xt_hbm, w_hbm, out_hbm,
          s_ref, w_ref, xa_ref, xb_ref, acc_ref, c_ref,
          sem_a, sem_b):
    b = lax.axis_index("s") * _NC + lax.axis_index("c")

    big16 = jnp.full((16,), 1e38, jnp.float32)

    def init_body(g, _):
        s_ref[pl.ds(g * 16, 16)] = big16
        return 0
    lax.fori_loop(0, _N // 16, init_body, 0)

    _chunk_copy(xt_hbm, b, 0, xa_ref, sem_a).start()

    lane = lax.iota(jnp.int32, 16)

    def step(t, _):
        pltpu.sync_copy(w_hbm.at[b, t], w_ref)

        def abody(g, carry):
            mv, iv = carry
            p = s_ref[pl.ds(g * 16, 16)] * w_ref[pl.ds(g * 16, 16)]
            upd = p > mv
            return (jnp.where(upd, p, mv),
                    jnp.where(upd, g * 16 + lane, iv))

        mv, iv = lax.fori_loop(
            0, _N // 16, abody,
            (jnp.full((16,), -1.0, jnp.float32), jnp.zeros((16,), jnp.int32)))
        m = jnp.max(mv)
        idx = jnp.min(jnp.where(mv == m, iv, jnp.int32(2 ** 30)))

        pltpu.sync_copy(x_hbm.at[b, idx], c_ref)
        pltpu.sync_copy(c_ref, out_hbm.at[b, t])

        @pl.when(t < _K - 1)
        def _update():
            def pair(g, _):
                c0 = 2 * g
                c1 = 2 * g + 1
                nxt = (2 * g + 2) % _NCH
                _chunk_copy(xt_hbm, b, c1, xb_ref, sem_b).start()
                _chunk_copy(xt_hbm, b, c0, xa_ref, sem_a).wait()
                _process_chunk(xa_ref, c0 * _TK, c_ref, s_ref, acc_ref)
                _chunk_copy(xt_hbm, b, nxt, xa_ref, sem_a).start()
                _chunk_copy(xt_hbm, b, c1, xb_ref, sem_b).wait()
                _process_chunk(xb_ref, c1 * _TK, c_ref, s_ref, acc_ref)
                return 0
            lax.fori_loop(0, _NCH // 2, pair, 0)

        return 0

    lax.fori_loop(0, _K, step, 0)
    _chunk_copy(xt_hbm, b, 0, xa_ref, sem_a).wait()


@functools.partial(
    pl.kernel,
    mesh=plsc.VectorSubcoreMesh(core_axis_name="c", subcore_axis_name="s"),
    compiler_params=pltpu.CompilerParams(needs_layout_passes=False),
    out_type=jax.ShapeDtypeStruct((_B, _K, _D), jnp.float32),
    scratch_types=[
        pltpu.VMEM((_N,), jnp.float32),
        pltpu.VMEM((_N,), jnp.float32),
        pltpu.VMEM((_D, _TK), jnp.float32),
        pltpu.VMEM((_D, _TK), jnp.float32),
        pltpu.VMEM((_TK,), jnp.float32),
        pltpu.VMEM((_D,), jnp.float32),
        pltpu.SemaphoreType.DMA,
        pltpu.SemaphoreType.DMA,
    ],
)
def _diverse_sc(x_hbm, xt_hbm, w_hbm, out_hbm,
                s_ref, w_ref, xa_ref, xb_ref, acc_ref, c_ref,
                sem_a, sem_b):
    _body(x_hbm, xt_hbm, w_hbm, out_hbm,
          s_ref, w_ref, xa_ref, xb_ref, acc_ref, c_ref,
          sem_a, sem_b)


def kernel(x):
    w = _build_sampling_consts()
    xt = jnp.swapaxes(x, 1, 2)
    tokens = _diverse_sc(x, xt, w)
    return tokens, jnp.float32(0.0)
